# baseline (speedup 1.0000x reference)
import functools

import jax
import jax.numpy as jnp
from jax import lax
from jax.experimental import pallas as pl
from jax.experimental.pallas import tpu as pltpu
from jax.experimental.pallas import tpu_sc as plsc

N, D, H, E = 8192, 1024, 512, 8
A = 2 * N
B = 512
Q = A + E * B
NBLK = Q // B

NC, NS = 2, 16
NW = NC * NS
WB = 128


def _bdot(a, b):
    return jnp.dot(a.astype(jnp.bfloat16), b.astype(jnp.bfloat16),
                   preferred_element_type=jnp.float32)


_HI_MASK = 0xFFFF0000


def _pack_rows(v):
    r = v.astype(jnp.bfloat16).astype(jnp.float32)
    hw = r.shape[-1] // 2
    u0 = lax.bitcast_convert_type(r[:, :hw], jnp.uint32) >> 16
    u1 = lax.bitcast_convert_type(r[:, hw:], jnp.uint32) & jnp.uint32(_HI_MASK)
    return lax.bitcast_convert_type(u0 | u1, jnp.int32)


def _unpack_rows(p):
    pu = lax.bitcast_convert_type(p, jnp.uint32)
    b0 = lax.bitcast_convert_type(pu << 16, jnp.float32)
    b1 = lax.bitcast_convert_type(pu & jnp.uint32(_HI_MASK), jnp.float32)
    return b0, b1


def _gate_kernel(x_ref, gW1_ref, gb1_ref, gW2_ref, gb2_ref, gW3_ref, gb3_ref,
                 probs_ref, i1_ref, i2_ref, w1_ref, w2_ref, xb_ref):
    x = x_ref[...]
    xb_ref[...] = _pack_rows(x)
    h = jnp.maximum(_bdot(x, gW1_ref[...]) + gb1_ref[...], 0.0)
    h = jnp.maximum(_bdot(h, gW2_ref[...]) + gb2_ref[...], 0.0)
    logits = _bdot(h, gW3_ref[...]) + gb3_ref[...]
    m = jnp.max(logits, axis=-1, keepdims=True)
    ex = jnp.exp(logits - m)
    p = ex / jnp.sum(ex, axis=-1, keepdims=True)
    probs_ref[...] = p

    col = jax.lax.broadcasted_iota(jnp.int32, p.shape, 1)
    v1 = jnp.max(p, axis=-1, keepdims=True)
    i1 = jnp.min(jnp.where(p == v1, col, E), axis=-1, keepdims=True)
    pm = jnp.where(col == i1, -1.0, p)
    v2 = jnp.max(pm, axis=-1, keepdims=True)
    i2 = jnp.min(jnp.where(pm == v2, col, E), axis=-1, keepdims=True)
    s = v1 + v2
    i1_ref[...] = i1
    i2_ref[...] = i2
    w1_ref[...] = v1 / s
    w2_ref[...] = v2 / s


def _fs_kernel(x_ref, fW1_ref, fb1_ref, fW2_ref, fb2_ref,
               sW1_ref, sb1_ref, sW2_ref, sb2_ref, fiber_ref, smooth_ref):
    x = x_ref[...]
    fh = jnp.maximum(_bdot(x, fW1_ref[...]) + fb1_ref[...], 0.0)
    fiber_ref[...] = jnp.tanh(_bdot(fh, fW2_ref[...]) + fb2_ref[...])
    sh = jnp.maximum(_bdot(x, sW1_ref[...]) + sb1_ref[...], 0.0)
    smooth_ref[...] = jax.nn.sigmoid(_bdot(sh, sW2_ref[...]) + sb2_ref[...])


def _run_fs(x, fW1, fb1, fW2, fb2, sW1, sb1, sW2, sb2):
    TB = 1024
    r2 = lambda b: b.reshape(1, -1)
    full = lambda shape: pl.BlockSpec(shape, lambda i: tuple(0 for _ in shape))
    return pl.pallas_call(
        _fs_kernel,
        grid=(N // TB,),
        in_specs=[
            pl.BlockSpec((TB, D), lambda i: (i, 0)),
            full((D, H)), full((1, H)), full((H, 1)), full((1, 1)),
            full((D, H)), full((1, H)), full((H, 1)), full((1, 1)),
        ],
        out_specs=[
            pl.BlockSpec((TB, 1), lambda i: (i, 0)),
            pl.BlockSpec((TB, 1), lambda i: (i, 0)),
        ],
        out_shape=[
            jax.ShapeDtypeStruct((N, 1), jnp.float32),
            jax.ShapeDtypeStruct((N, 1), jnp.float32),
        ],
        compiler_params=pltpu.CompilerParams(
            dimension_semantics=("parallel",)),
    )(x, fW1, r2(fb1), fW2, r2(fb2), sW1, r2(sb1), sW2, r2(sb2))


def _run_gate(x, gW1, gb1, gW2, gb2, gW3, gb3):
    TB = 1024
    r2 = lambda b: b.reshape(1, -1)
    full = lambda shape: pl.BlockSpec(shape, lambda i: tuple(0 for _ in shape))
    return pl.pallas_call(
        _gate_kernel,
        grid=(N // TB,),
        in_specs=[
            pl.BlockSpec((TB, D), lambda i: (i, 0)),
            full((D, H)), full((1, H)), full((H, H)), full((1, H)),
            full((H, E)), full((1, E)),
        ],
        out_specs=[
            pl.BlockSpec((TB, E), lambda i: (i, 0)),
            pl.BlockSpec((TB, 1), lambda i: (i, 0)),
            pl.BlockSpec((TB, 1), lambda i: (i, 0)),
            pl.BlockSpec((TB, 1), lambda i: (i, 0)),
            pl.BlockSpec((TB, 1), lambda i: (i, 0)),
            pl.BlockSpec((TB, D // 2), lambda i: (i, 0)),
        ],
        out_shape=[
            jax.ShapeDtypeStruct((N, E), jnp.float32),
            jax.ShapeDtypeStruct((N, 1), jnp.int32),
            jax.ShapeDtypeStruct((N, 1), jnp.int32),
            jax.ShapeDtypeStruct((N, 1), jnp.float32),
            jax.ShapeDtypeStruct((N, 1), jnp.float32),
            jax.ShapeDtypeStruct((N, D // 2), jnp.int32),
        ],
        compiler_params=pltpu.CompilerParams(
            dimension_semantics=("parallel",)),
    )(x, gW1, r2(gb1), gW2, r2(gb2), gW3, r2(gb3))


def _route(i1, i2, w1, w2):
    ids = jnp.concatenate([i1[:, 0], i2[:, 0]])
    onehot = (ids[:, None] == jnp.arange(E, dtype=jnp.int32)[None, :])
    oh = onehot.astype(jnp.int32)
    ranks_inc = jnp.cumsum(oh, axis=0)
    counts = ranks_inc[-1]
    nb = (counts + B - 1) // B
    ends = jnp.cumsum(nb)
    po = jnp.concatenate([jnp.zeros((1,), jnp.int32),
                          (ends[:-1] * B).astype(jnp.int32)])
    pp = jnp.sum(jnp.where(onehot, ranks_inc - oh + po[None, :], 0),
                 axis=1).astype(jnp.int32)
    bidx = jnp.arange(NBLK, dtype=jnp.int32)
    be = jnp.minimum(
        jnp.sum((bidx[:, None] >= ends[None, :]).astype(jnp.int32), axis=1),
        E - 1).astype(jnp.int32)
    return pp, be


def _sc_dispatch(xb, pp):
    mesh = plsc.VectorSubcoreMesh(core_axis_name="c", subcore_axis_name="s")

    @functools.partial(
        pl.kernel, mesh=mesh,
        out_type=jax.ShapeDtypeStruct((Q, D // 2), jnp.int32),
        scratch_types=[pltpu.VMEM((WB,), jnp.int32),
                       pltpu.VMEM((WB, D // 2), jnp.int32)],
    )
    def dispatch(x_hbm, pp_hbm, xg_hbm, idx_v, rows_v):
        wid = lax.axis_index("s") * NC + lax.axis_index("c")
        base = wid * (A // NW)

        @pl.loop(0, A // NW, step=WB)
        def _(off):
            j = base + off
            xoff = lax.rem(j, N)
            pltpu.sync_copy(pp_hbm.at[pl.ds(j, WB)], idx_v)
            pltpu.sync_copy(x_hbm.at[pl.ds(xoff, WB)], rows_v)
            pltpu.sync_copy(rows_v, xg_hbm.at[idx_v])

    return dispatch(xb, pp)


def _expert_kernel(be_ref, xg_ref, eW1a_ref, eW1b_ref, eb1_ref, eW2_ref,
                   eb2_ref, eW3_ref, eb3_ref, ys_ref):
    b0, b1 = _unpack_rows(xg_ref[...])
    h = jnp.maximum(_bdot(b0, eW1a_ref[0]) + _bdot(b1, eW1b_ref[0])
                    + eb1_ref[0], 0.0)
    h = jnp.maximum(_bdot(h, eW2_ref[0]) + eb2_ref[0], 0.0)
    o = _bdot(h, eW3_ref[0]) + eb3_ref[0]
    ys_ref[...] = _pack_rows(o)


def _run_experts(xg, be, eW1, eb1, eW2, eb2, eW3, eb3):
    grid_spec = pltpu.PrefetchScalarGridSpec(
        num_scalar_prefetch=1,
        grid=(NBLK,),
        in_specs=[
            pl.BlockSpec((B, D // 2), lambda b, be: (b, 0)),
            pl.BlockSpec((1, D // 2, H), lambda b, be: (be[b], 0, 0)),
            pl.BlockSpec((1, D // 2, H), lambda b, be: (be[b], 1, 0)),
            pl.BlockSpec((1, 1, H), lambda b, be: (be[b], 0, 0)),
            pl.BlockSpec((1, H, H), lambda b, be: (be[b], 0, 0)),
            pl.BlockSpec((1, 1, H), lambda b, be: (be[b], 0, 0)),
            pl.BlockSpec((1, H, D), lambda b, be: (be[b], 0, 0)),
            pl.BlockSpec((1, 1, D), lambda b, be: (be[b], 0, 0)),
        ],
        out_specs=pl.BlockSpec((B, D // 2), lambda b, be: (b, 0)),
    )
    return pl.pallas_call(
        _expert_kernel,
        grid_spec=grid_spec,
        out_shape=jax.ShapeDtypeStruct((Q, D // 2), jnp.int32),
        compiler_params=pltpu.CompilerParams(
            dimension_semantics=("arbitrary",)),
    )(be, xg, eW1, eW1, eb1[:, None, :], eW2, eb2[:, None, :],
      eW3, eb3[:, None, :])


def _sc_combine(ys, pp):
    mesh = plsc.VectorSubcoreMesh(core_axis_name="c", subcore_axis_name="s")

    @functools.partial(
        pl.kernel, mesh=mesh,
        out_type=jax.ShapeDtypeStruct((A, D // 2), jnp.int32),
        scratch_types=[pltpu.VMEM((WB,), jnp.int32),
                       pltpu.VMEM((WB, D // 2), jnp.int32),
                       pltpu.SemaphoreType.DMA],
    )
    def combine(ys_hbm, pp_hbm, g_hbm, idx_v, rows_v, sem):
        wid = lax.axis_index("s") * NC + lax.axis_index("c")
        base = wid * (A // NW)

        @pl.loop(0, A // NW, step=WB)
        def _(off):
            j = base + off
            pltpu.sync_copy(pp_hbm.at[pl.ds(j, WB)], idx_v)
            pltpu.async_copy(ys_hbm.at[idx_v], rows_v, sem).wait()
            pltpu.sync_copy(rows_v, g_hbm.at[pl.ds(j, WB)])

    return combine(ys, pp)


def _add_kernel(g0_ref, g1_ref, w1_ref, w2_ref, out_ref):
    a0, a1 = _unpack_rows(g0_ref[...])
    b0, b1 = _unpack_rows(g1_ref[...])
    w1 = w1_ref[...]
    w2 = w2_ref[...]
    out_ref[:, :D // 2] = a0 * w1 + b0 * w2
    out_ref[:, D // 2:] = a1 * w1 + b1 * w2


def _run_add(g, w1, w2):
    TB = 1024
    nb = N // TB
    return pl.pallas_call(
        _add_kernel,
        grid=(nb,),
        in_specs=[
            pl.BlockSpec((TB, D // 2), lambda i: (i, 0)),
            pl.BlockSpec((TB, D // 2), lambda i: (i + nb, 0)),
            pl.BlockSpec((TB, 1), lambda i: (i, 0)),
            pl.BlockSpec((TB, 1), lambda i: (i, 0)),
        ],
        out_specs=pl.BlockSpec((TB, D), lambda i: (i, 0)),
        out_shape=jax.ShapeDtypeStruct((N, D), jnp.float32),
        compiler_params=pltpu.CompilerParams(
            dimension_semantics=("parallel",)),
    )(g, g, w1, w2)


def kernel(x, eW1, eb1, eW2, eb2, eW3, eb3, gW1, gb1, gW2, gb2, gW3, gb3,
           fW1, fb1, fW2, fb2, sW1, sb1, sW2, sb2):
    (gate_probs, i1, i2, w1, w2, xb) = _run_gate(
        x, gW1, gb1, gW2, gb2, gW3, gb3)
    fiber, smooth = _run_fs(x, fW1, fb1, fW2, fb2, sW1, sb1, sW2, sb2)

    pp, be = _route(i1, i2, w1, w2)
    xg = _sc_dispatch(xb, pp)
    ys = _run_experts(xg, be, eW1, eb1, eW2, eb2, eW3, eb3)
    g = _sc_combine(ys, pp)
    out = _run_add(g, w1, w2)
    return (out, gate_probs, fiber, smooth)


# speedup vs baseline: 1.0015x; 1.0015x over previous
import functools

import jax
import jax.numpy as jnp
from jax import lax
from jax.experimental import pallas as pl
from jax.experimental.pallas import tpu as pltpu
from jax.experimental.pallas import tpu_sc as plsc

N, D, H, E = 8192, 1024, 512, 8
A = 2 * N
B = 512
Q = A + E * B
NBLK = Q // B

NC, NS = 2, 16
NW = NC * NS
WB = 128


def _bdot(a, b):
    return jnp.dot(a.astype(jnp.bfloat16), b.astype(jnp.bfloat16),
                   preferred_element_type=jnp.float32)


_HI_MASK = 0xFFFF0000


def _pack_rows(v):
    r = v.astype(jnp.bfloat16).astype(jnp.float32)
    hw = r.shape[-1] // 2
    u0 = lax.bitcast_convert_type(r[:, :hw], jnp.uint32) >> 16
    u1 = lax.bitcast_convert_type(r[:, hw:], jnp.uint32) & jnp.uint32(_HI_MASK)
    return lax.bitcast_convert_type(u0 | u1, jnp.int32)


def _unpack_rows(p):
    pu = lax.bitcast_convert_type(p, jnp.uint32)
    b0 = lax.bitcast_convert_type(pu << 16, jnp.float32)
    b1 = lax.bitcast_convert_type(pu & jnp.uint32(_HI_MASK), jnp.float32)
    return b0, b1


def _gate_kernel(x_ref, gW1_ref, gb1_ref, gW2_ref, gb2_ref, gW3_ref, gb3_ref,
                 fW1_ref, fb1_ref, fW2_ref, fb2_ref,
                 sW1_ref, sb1_ref, sW2_ref, sb2_ref,
                 probs_ref, i1_ref, i2_ref, w1_ref, w2_ref,
                 fiber_ref, smooth_ref, xb_ref):
    x = x_ref[...]
    xb_ref[...] = _pack_rows(x)
    h = jnp.maximum(_bdot(x, gW1_ref[...]) + gb1_ref[...], 0.0)
    h = jnp.maximum(_bdot(h, gW2_ref[...]) + gb2_ref[...], 0.0)
    logits = _bdot(h, gW3_ref[...]) + gb3_ref[...]
    m = jnp.max(logits, axis=-1, keepdims=True)
    ex = jnp.exp(logits - m)
    p = ex / jnp.sum(ex, axis=-1, keepdims=True)
    probs_ref[...] = p

    col = jax.lax.broadcasted_iota(jnp.int32, p.shape, 1)
    v1 = jnp.max(p, axis=-1, keepdims=True)
    i1 = jnp.min(jnp.where(p == v1, col, E), axis=-1, keepdims=True)
    pm = jnp.where(col == i1, -1.0, p)
    v2 = jnp.max(pm, axis=-1, keepdims=True)
    i2 = jnp.min(jnp.where(pm == v2, col, E), axis=-1, keepdims=True)
    s = v1 + v2
    i1_ref[...] = i1
    i2_ref[...] = i2
    w1_ref[...] = v1 / s
    w2_ref[...] = v2 / s

    fh = jnp.maximum(_bdot(x, fW1_ref[...]) + fb1_ref[...], 0.0)
    fiber_ref[...] = jnp.tanh(_bdot(fh, fW2_ref[...]) + fb2_ref[...])
    sh = jnp.maximum(_bdot(x, sW1_ref[...]) + sb1_ref[...], 0.0)
    smooth_ref[...] = jax.nn.sigmoid(_bdot(sh, sW2_ref[...]) + sb2_ref[...])


def _run_gate(x, gW1, gb1, gW2, gb2, gW3, gb3, fW1, fb1, fW2, fb2,
              sW1, sb1, sW2, sb2):
    TB = 2048
    r2 = lambda b: b.reshape(1, -1)
    full = lambda shape: pl.BlockSpec(shape, lambda i: tuple(0 for _ in shape))
    return pl.pallas_call(
        _gate_kernel,
        grid=(N // TB,),
        in_specs=[
            pl.BlockSpec((TB, D), lambda i: (i, 0)),
            full((D, H)), full((1, H)), full((H, H)), full((1, H)),
            full((H, E)), full((1, E)),
            full((D, H)), full((1, H)), full((H, 1)), full((1, 1)),
            full((D, H)), full((1, H)), full((H, 1)), full((1, 1)),
        ],
        out_specs=[
            pl.BlockSpec((TB, E), lambda i: (i, 0)),
            pl.BlockSpec((TB, 1), lambda i: (i, 0)),
            pl.BlockSpec((TB, 1), lambda i: (i, 0)),
            pl.BlockSpec((TB, 1), lambda i: (i, 0)),
            pl.BlockSpec((TB, 1), lambda i: (i, 0)),
            pl.BlockSpec((TB, 1), lambda i: (i, 0)),
            pl.BlockSpec((TB, 1), lambda i: (i, 0)),
            pl.BlockSpec((TB, D // 2), lambda i: (i, 0)),
        ],
        out_shape=[
            jax.ShapeDtypeStruct((N, E), jnp.float32),
            jax.ShapeDtypeStruct((N, 1), jnp.int32),
            jax.ShapeDtypeStruct((N, 1), jnp.int32),
            jax.ShapeDtypeStruct((N, 1), jnp.float32),
            jax.ShapeDtypeStruct((N, 1), jnp.float32),
            jax.ShapeDtypeStruct((N, 1), jnp.float32),
            jax.ShapeDtypeStruct((N, 1), jnp.float32),
            jax.ShapeDtypeStruct((N, D // 2), jnp.int32),
        ],
        compiler_params=pltpu.CompilerParams(
            dimension_semantics=("parallel",)),
    )(x, gW1, r2(gb1), gW2, r2(gb2), gW3, r2(gb3),
      fW1, r2(fb1), fW2, r2(fb2), sW1, r2(sb1), sW2, r2(sb2))


def _route(i1, i2, w1, w2):
    ids = jnp.concatenate([i1[:, 0], i2[:, 0]])
    onehot = (ids[:, None] == jnp.arange(E, dtype=jnp.int32)[None, :])
    oh = onehot.astype(jnp.int32)
    ranks_inc = jnp.cumsum(oh, axis=0)
    counts = ranks_inc[-1]
    nb = (counts + B - 1) // B
    ends = jnp.cumsum(nb)
    po = jnp.concatenate([jnp.zeros((1,), jnp.int32),
                          (ends[:-1] * B).astype(jnp.int32)])
    pp = jnp.sum(jnp.where(onehot, ranks_inc - oh + po[None, :], 0),
                 axis=1).astype(jnp.int32)
    bidx = jnp.arange(NBLK, dtype=jnp.int32)
    be = jnp.minimum(
        jnp.sum((bidx[:, None] >= ends[None, :]).astype(jnp.int32), axis=1),
        E - 1).astype(jnp.int32)
    return pp, be


def _sc_dispatch(xb, pp):
    mesh = plsc.VectorSubcoreMesh(core_axis_name="c", subcore_axis_name="s")

    @functools.partial(
        pl.kernel, mesh=mesh,
        out_type=jax.ShapeDtypeStruct((Q, D // 2), jnp.int32),
        scratch_types=[pltpu.VMEM((WB,), jnp.int32),
                       pltpu.VMEM((WB, D // 2), jnp.int32)],
    )
    def dispatch(x_hbm, pp_hbm, xg_hbm, idx_v, rows_v):
        wid = lax.axis_index("s") * NC + lax.axis_index("c")
        base = wid * (A // NW)

        @pl.loop(0, A // NW, step=WB)
        def _(off):
            j = base + off
            xoff = lax.rem(j, N)
            pltpu.sync_copy(pp_hbm.at[pl.ds(j, WB)], idx_v)
            pltpu.sync_copy(x_hbm.at[pl.ds(xoff, WB)], rows_v)
            pltpu.sync_copy(rows_v, xg_hbm.at[idx_v])

    return dispatch(xb, pp)


def _expert_kernel(be_ref, xg_ref, eW1a_ref, eW1b_ref, eb1_ref, eW2_ref,
                   eb2_ref, eW3_ref, eb3_ref, ys_ref,
                   w1a_c, w1b_c, w2_c, w3_c):
    b = pl.program_id(0)
    prev = be_ref[jnp.maximum(b - 1, 0)]

    @pl.when((b == 0) | (be_ref[b] != prev))
    def _():
        w1a_c[...] = eW1a_ref[0].astype(jnp.bfloat16)
        w1b_c[...] = eW1b_ref[0].astype(jnp.bfloat16)
        w2_c[...] = eW2_ref[0].astype(jnp.bfloat16)
        w3_c[...] = eW3_ref[0].astype(jnp.bfloat16)

    b0, b1 = _unpack_rows(xg_ref[...])
    h = jnp.maximum(_bdot(b0, w1a_c[...]) + _bdot(b1, w1b_c[...])
                    + eb1_ref[0], 0.0)
    h = jnp.maximum(_bdot(h, w2_c[...]) + eb2_ref[0], 0.0)
    o = _bdot(h, w3_c[...]) + eb3_ref[0]
    ys_ref[...] = _pack_rows(o)


def _run_experts(xg, be, eW1, eb1, eW2, eb2, eW3, eb3):
    grid_spec = pltpu.PrefetchScalarGridSpec(
        num_scalar_prefetch=1,
        grid=(NBLK,),
        in_specs=[
            pl.BlockSpec((B, D // 2), lambda b, be: (b, 0)),
            pl.BlockSpec((1, D // 2, H), lambda b, be: (be[b], 0, 0)),
            pl.BlockSpec((1, D // 2, H), lambda b, be: (be[b], 1, 0)),
            pl.BlockSpec((1, 1, H), lambda b, be: (be[b], 0, 0)),
            pl.BlockSpec((1, H, H), lambda b, be: (be[b], 0, 0)),
            pl.BlockSpec((1, 1, H), lambda b, be: (be[b], 0, 0)),
            pl.BlockSpec((1, H, D), lambda b, be: (be[b], 0, 0)),
            pl.BlockSpec((1, 1, D), lambda b, be: (be[b], 0, 0)),
        ],
        out_specs=pl.BlockSpec((B, D // 2), lambda b, be: (b, 0)),
        scratch_shapes=[
            pltpu.VMEM((D // 2, H), jnp.bfloat16),
            pltpu.VMEM((D // 2, H), jnp.bfloat16),
            pltpu.VMEM((H, H), jnp.bfloat16),
            pltpu.VMEM((H, D), jnp.bfloat16),
        ],
    )
    return pl.pallas_call(
        _expert_kernel,
        grid_spec=grid_spec,
        out_shape=jax.ShapeDtypeStruct((Q, D // 2), jnp.int32),
        compiler_params=pltpu.CompilerParams(
            dimension_semantics=("arbitrary",)),
    )(be, xg, eW1, eW1, eb1[:, None, :], eW2, eb2[:, None, :],
      eW3, eb3[:, None, :])


def _sc_combine(ys, pp):
    mesh = plsc.VectorSubcoreMesh(core_axis_name="c", subcore_axis_name="s")

    @functools.partial(
        pl.kernel, mesh=mesh,
        out_type=jax.ShapeDtypeStruct((A, D // 2), jnp.int32),
        scratch_types=[pltpu.VMEM((WB,), jnp.int32),
                       pltpu.VMEM((WB, D // 2), jnp.int32),
                       pltpu.SemaphoreType.DMA],
    )
    def combine(ys_hbm, pp_hbm, g_hbm, idx_v, rows_v, sem):
        wid = lax.axis_index("s") * NC + lax.axis_index("c")
        base = wid * (A // NW)

        @pl.loop(0, A // NW, step=WB)
        def _(off):
            j = base + off
            pltpu.sync_copy(pp_hbm.at[pl.ds(j, WB)], idx_v)
            pltpu.async_copy(ys_hbm.at[idx_v], rows_v, sem).wait()
            pltpu.sync_copy(rows_v, g_hbm.at[pl.ds(j, WB)])

    return combine(ys, pp)


def _add_kernel(g0_ref, g1_ref, w1_ref, w2_ref, out_ref):
    a0, a1 = _unpack_rows(g0_ref[...])
    b0, b1 = _unpack_rows(g1_ref[...])
    w1 = w1_ref[...]
    w2 = w2_ref[...]
    out_ref[:, :D // 2] = a0 * w1 + b0 * w2
    out_ref[:, D // 2:] = a1 * w1 + b1 * w2


def _run_add(g, w1, w2):
    TB = 1024
    nb = N // TB
    return pl.pallas_call(
        _add_kernel,
        grid=(nb,),
        in_specs=[
            pl.BlockSpec((TB, D // 2), lambda i: (i, 0)),
            pl.BlockSpec((TB, D // 2), lambda i: (i + nb, 0)),
            pl.BlockSpec((TB, 1), lambda i: (i, 0)),
            pl.BlockSpec((TB, 1), lambda i: (i, 0)),
        ],
        out_specs=pl.BlockSpec((TB, D), lambda i: (i, 0)),
        out_shape=jax.ShapeDtypeStruct((N, D), jnp.float32),
        compiler_params=pltpu.CompilerParams(
            dimension_semantics=("parallel",)),
    )(g, g, w1, w2)


def kernel(x, eW1, eb1, eW2, eb2, eW3, eb3, gW1, gb1, gW2, gb2, gW3, gb3,
           fW1, fb1, fW2, fb2, sW1, sb1, sW2, sb2):
    (gate_probs, i1, i2, w1, w2, fiber, smooth, xb) = _run_gate(
        x, gW1, gb1, gW2, gb2, gW3, gb3, fW1, fb1, fW2, fb2,
        sW1, sb1, sW2, sb2)

    pp, be = _route(i1, i2, w1, w2)
    xg = _sc_dispatch(xb, pp)
    ys = _run_experts(xg, be, eW1, eb1, eW2, eb2, eW3, eb3)
    g = _sc_combine(ys, pp)
    out = _run_add(g, w1, w2)
    return (out, gate_probs, fiber, smooth)


# speedup vs baseline: 1.0436x; 1.0420x over previous
import functools

import jax
import jax.numpy as jnp
from jax import lax
from jax.experimental import pallas as pl
from jax.experimental.pallas import tpu as pltpu
from jax.experimental.pallas import tpu_sc as plsc

N, D, H, E = 8192, 1024, 512, 8
A = 2 * N
B = 512
Q = A + E * B
NBLK = Q // B

NC, NS = 2, 16
NW = NC * NS
WB = 128


def _bdot(a, b):
    return jnp.dot(a.astype(jnp.bfloat16), b.astype(jnp.bfloat16),
                   preferred_element_type=jnp.float32)


_HI_MASK = 0xFFFF0000


def _pack_rows(v):
    r = v.astype(jnp.bfloat16).astype(jnp.float32)
    hw = r.shape[-1] // 2
    u0 = lax.bitcast_convert_type(r[:, :hw], jnp.uint32) >> 16
    u1 = lax.bitcast_convert_type(r[:, hw:], jnp.uint32) & jnp.uint32(_HI_MASK)
    return lax.bitcast_convert_type(u0 | u1, jnp.int32)


def _unpack_rows(p):
    pu = lax.bitcast_convert_type(p, jnp.uint32)
    b0 = lax.bitcast_convert_type(pu << 16, jnp.float32)
    b1 = lax.bitcast_convert_type(pu & jnp.uint32(_HI_MASK), jnp.float32)
    return b0, b1


def _gate_kernel(x_ref, gW1_ref, gb1_ref, gW2_ref, gb2_ref, gW3_ref, gb3_ref,
                 fW1_ref, fb1_ref, fW2_ref, fb2_ref,
                 sW1_ref, sb1_ref, sW2_ref, sb2_ref,
                 probs_ref, i1_ref, i2_ref, w1_ref, w2_ref,
                 fiber_ref, smooth_ref, xb_ref):
    x = x_ref[...]
    xb_ref[...] = _pack_rows(x)
    h = jnp.maximum(_bdot(x, gW1_ref[...]) + gb1_ref[...], 0.0)
    h = jnp.maximum(_bdot(h, gW2_ref[...]) + gb2_ref[...], 0.0)
    logits = _bdot(h, gW3_ref[...]) + gb3_ref[...]
    m = jnp.max(logits, axis=-1, keepdims=True)
    ex = jnp.exp(logits - m)
    p = ex / jnp.sum(ex, axis=-1, keepdims=True)
    probs_ref[...] = p

    col = jax.lax.broadcasted_iota(jnp.int32, p.shape, 1)
    v1 = jnp.max(p, axis=-1, keepdims=True)
    i1 = jnp.min(jnp.where(p == v1, col, E), axis=-1, keepdims=True)
    pm = jnp.where(col == i1, -1.0, p)
    v2 = jnp.max(pm, axis=-1, keepdims=True)
    i2 = jnp.min(jnp.where(pm == v2, col, E), axis=-1, keepdims=True)
    s = v1 + v2
    i1_ref[...] = i1
    i2_ref[...] = i2
    w1_ref[...] = v1 / s
    w2_ref[...] = v2 / s

    fh = jnp.maximum(_bdot(x, fW1_ref[...]) + fb1_ref[...], 0.0)
    fiber_ref[...] = jnp.tanh(_bdot(fh, fW2_ref[...]) + fb2_ref[...])
    sh = jnp.maximum(_bdot(x, sW1_ref[...]) + sb1_ref[...], 0.0)
    smooth_ref[...] = jax.nn.sigmoid(_bdot(sh, sW2_ref[...]) + sb2_ref[...])


def _run_gate(x, gW1, gb1, gW2, gb2, gW3, gb3, fW1, fb1, fW2, fb2,
              sW1, sb1, sW2, sb2):
    TB = 1024
    r2 = lambda b: b.reshape(1, -1)
    full = lambda shape: pl.BlockSpec(shape, lambda i: tuple(0 for _ in shape))
    return pl.pallas_call(
        _gate_kernel,
        grid=(N // TB,),
        in_specs=[
            pl.BlockSpec((TB, D), lambda i: (i, 0)),
            full((D, H)), full((1, H)), full((H, H)), full((1, H)),
            full((H, E)), full((1, E)),
            full((D, H)), full((1, H)), full((H, 1)), full((1, 1)),
            full((D, H)), full((1, H)), full((H, 1)), full((1, 1)),
        ],
        out_specs=[
            pl.BlockSpec((TB, E), lambda i: (i, 0)),
            pl.BlockSpec((TB, 1), lambda i: (i, 0)),
            pl.BlockSpec((TB, 1), lambda i: (i, 0)),
            pl.BlockSpec((TB, 1), lambda i: (i, 0)),
            pl.BlockSpec((TB, 1), lambda i: (i, 0)),
            pl.BlockSpec((TB, 1), lambda i: (i, 0)),
            pl.BlockSpec((TB, 1), lambda i: (i, 0)),
            pl.BlockSpec((TB, D // 2), lambda i: (i, 0)),
        ],
        out_shape=[
            jax.ShapeDtypeStruct((N, E), jnp.float32),
            jax.ShapeDtypeStruct((N, 1), jnp.int32),
            jax.ShapeDtypeStruct((N, 1), jnp.int32),
            jax.ShapeDtypeStruct((N, 1), jnp.float32),
            jax.ShapeDtypeStruct((N, 1), jnp.float32),
            jax.ShapeDtypeStruct((N, 1), jnp.float32),
            jax.ShapeDtypeStruct((N, 1), jnp.float32),
            jax.ShapeDtypeStruct((N, D // 2), jnp.int32),
        ],
        compiler_params=pltpu.CompilerParams(
            dimension_semantics=("parallel",)),
    )(x, gW1, r2(gb1), gW2, r2(gb2), gW3, r2(gb3),
      fW1, r2(fb1), fW2, r2(fb2), sW1, r2(sb1), sW2, r2(sb2))


def _route(i1, i2, w1, w2):
    ids = jnp.concatenate([i1[:, 0], i2[:, 0]])
    onehot = (ids[:, None] == jnp.arange(E, dtype=jnp.int32)[None, :])
    oh = onehot.astype(jnp.int32)
    ranks_inc = jnp.cumsum(oh, axis=0)
    counts = ranks_inc[-1]
    nb = (counts + B - 1) // B
    ends = jnp.cumsum(nb)
    po = jnp.concatenate([jnp.zeros((1,), jnp.int32),
                          (ends[:-1] * B).astype(jnp.int32)])
    pp = jnp.sum(jnp.where(onehot, ranks_inc - oh + po[None, :], 0),
                 axis=1).astype(jnp.int32)
    bidx = jnp.arange(NBLK, dtype=jnp.int32)
    be = jnp.minimum(
        jnp.sum((bidx[:, None] >= ends[None, :]).astype(jnp.int32), axis=1),
        E - 1).astype(jnp.int32)
    return pp, be


def _sc_dispatch(xb, pp):
    mesh = plsc.VectorSubcoreMesh(core_axis_name="c", subcore_axis_name="s")

    @functools.partial(
        pl.kernel, mesh=mesh,
        out_type=jax.ShapeDtypeStruct((Q, D // 2), jnp.int32),
        scratch_types=[pltpu.VMEM((WB,), jnp.int32),
                       pltpu.VMEM((WB, D // 2), jnp.int32)],
    )
    def dispatch(x_hbm, pp_hbm, xg_hbm, idx_v, rows_v):
        wid = lax.axis_index("s") * NC + lax.axis_index("c")
        base = wid * (N // NW)

        @pl.loop(0, N // NW, step=WB)
        def _(off):
            t = base + off
            pltpu.sync_copy(x_hbm.at[pl.ds(t, WB)], rows_v)
            pltpu.sync_copy(pp_hbm.at[pl.ds(t, WB)], idx_v)
            pltpu.sync_copy(rows_v, xg_hbm.at[idx_v])
            pltpu.sync_copy(pp_hbm.at[pl.ds(N + t, WB)], idx_v)
            pltpu.sync_copy(rows_v, xg_hbm.at[idx_v])

    return dispatch(xb, pp)


def _expert_kernel(be_ref, xg_ref, eW1a_ref, eW1b_ref, eb1_ref, eW2_ref,
                   eb2_ref, eW3_ref, eb3_ref, ys_ref):
    b0, b1 = _unpack_rows(xg_ref[...])
    h = jnp.maximum(_bdot(b0, eW1a_ref[0]) + _bdot(b1, eW1b_ref[0])
                    + eb1_ref[0], 0.0)
    h = jnp.maximum(_bdot(h, eW2_ref[0]) + eb2_ref[0], 0.0)
    o = _bdot(h, eW3_ref[0]) + eb3_ref[0]
    ys_ref[...] = _pack_rows(o)


def _run_experts(xg, be, eW1, eb1, eW2, eb2, eW3, eb3):
    grid_spec = pltpu.PrefetchScalarGridSpec(
        num_scalar_prefetch=1,
        grid=(NBLK,),
        in_specs=[
            pl.BlockSpec((B, D // 2), lambda b, be: (b, 0)),
            pl.BlockSpec((1, D // 2, H), lambda b, be: (be[b], 0, 0)),
            pl.BlockSpec((1, D // 2, H), lambda b, be: (be[b], 1, 0)),
            pl.BlockSpec((1, 1, H), lambda b, be: (be[b], 0, 0)),
            pl.BlockSpec((1, H, H), lambda b, be: (be[b], 0, 0)),
            pl.BlockSpec((1, 1, H), lambda b, be: (be[b], 0, 0)),
            pl.BlockSpec((1, H, D), lambda b, be: (be[b], 0, 0)),
            pl.BlockSpec((1, 1, D), lambda b, be: (be[b], 0, 0)),
        ],
        out_specs=pl.BlockSpec((B, D // 2), lambda b, be: (b, 0)),
    )
    return pl.pallas_call(
        _expert_kernel,
        grid_spec=grid_spec,
        out_shape=jax.ShapeDtypeStruct((Q, D // 2), jnp.int32),
        compiler_params=pltpu.CompilerParams(
            dimension_semantics=("arbitrary",)),
    )(be, xg, eW1, eW1, eb1[:, None, :], eW2, eb2[:, None, :],
      eW3, eb3[:, None, :])


def _sc_combine(ys, pp):
    mesh = plsc.VectorSubcoreMesh(core_axis_name="c", subcore_axis_name="s")

    @functools.partial(
        pl.kernel, mesh=mesh,
        out_type=jax.ShapeDtypeStruct((A, D // 2), jnp.int32),
        scratch_types=[pltpu.VMEM((WB,), jnp.int32),
                       pltpu.VMEM((WB, D // 2), jnp.int32),
                       pltpu.SemaphoreType.DMA],
    )
    def combine(ys_hbm, pp_hbm, g_hbm, idx_v, rows_v, sem):
        wid = lax.axis_index("s") * NC + lax.axis_index("c")
        base = wid * (A // NW)

        @pl.loop(0, A // NW, step=WB)
        def _(off):
            j = base + off
            pltpu.sync_copy(pp_hbm.at[pl.ds(j, WB)], idx_v)
            pltpu.async_copy(ys_hbm.at[idx_v], rows_v, sem).wait()
            pltpu.sync_copy(rows_v, g_hbm.at[pl.ds(j, WB)])

    return combine(ys, pp)


def _add_kernel(g0_ref, g1_ref, w1_ref, w2_ref, out_ref):
    a0, a1 = _unpack_rows(g0_ref[...])
    b0, b1 = _unpack_rows(g1_ref[...])
    w1 = w1_ref[...]
    w2 = w2_ref[...]
    out_ref[:, :D // 2] = a0 * w1 + b0 * w2
    out_ref[:, D // 2:] = a1 * w1 + b1 * w2


def _run_add(g, w1, w2):
    TB = 1024
    nb = N // TB
    return pl.pallas_call(
        _add_kernel,
        grid=(nb,),
        in_specs=[
            pl.BlockSpec((TB, D // 2), lambda i: (i, 0)),
            pl.BlockSpec((TB, D // 2), lambda i: (i + nb, 0)),
            pl.BlockSpec((TB, 1), lambda i: (i, 0)),
            pl.BlockSpec((TB, 1), lambda i: (i, 0)),
        ],
        out_specs=pl.BlockSpec((TB, D), lambda i: (i, 0)),
        out_shape=jax.ShapeDtypeStruct((N, D), jnp.float32),
        compiler_params=pltpu.CompilerParams(
            dimension_semantics=("parallel",)),
    )(g, g, w1, w2)


def kernel(x, eW1, eb1, eW2, eb2, eW3, eb3, gW1, gb1, gW2, gb2, gW3, gb3,
           fW1, fb1, fW2, fb2, sW1, sb1, sW2, sb2):
    (gate_probs, i1, i2, w1, w2, fiber, smooth, xb) = _run_gate(
        x, gW1, gb1, gW2, gb2, gW3, gb3, fW1, fb1, fW2, fb2,
        sW1, sb1, sW2, sb2)

    pp, be = _route(i1, i2, w1, w2)
    xg = _sc_dispatch(xb, pp)
    ys = _run_experts(xg, be, eW1, eb1, eW2, eb2, eW3, eb3)
    g = _sc_combine(ys, pp)
    out = _run_add(g, w1, w2)
    return (out, gate_probs, fiber, smooth)


# speedup vs baseline: 1.0457x; 1.0020x over previous
import functools

import jax
import jax.numpy as jnp
from jax import lax
from jax.experimental import pallas as pl
from jax.experimental.pallas import tpu as pltpu
from jax.experimental.pallas import tpu_sc as plsc

N, D, H, E = 8192, 1024, 512, 8
A = 2 * N
B = 512
Q = A + E * B
NBLK = Q // B

NC, NS = 2, 16
NW = NC * NS
WB = 128
CWB = 64


def _bdot(a, b):
    return jnp.dot(a.astype(jnp.bfloat16), b.astype(jnp.bfloat16),
                   preferred_element_type=jnp.float32)


_HI_MASK = 0xFFFF0000


def _pack_rows(v):
    r = v.astype(jnp.bfloat16).astype(jnp.float32)
    hw = r.shape[-1] // 2
    u0 = lax.bitcast_convert_type(r[:, :hw], jnp.uint32) >> 16
    u1 = lax.bitcast_convert_type(r[:, hw:], jnp.uint32) & jnp.uint32(_HI_MASK)
    return lax.bitcast_convert_type(u0 | u1, jnp.int32)


def _unpack_rows(p):
    pu = lax.bitcast_convert_type(p, jnp.uint32)
    b0 = lax.bitcast_convert_type(pu << 16, jnp.float32)
    b1 = lax.bitcast_convert_type(pu & jnp.uint32(_HI_MASK), jnp.float32)
    return b0, b1


def _gate_kernel(x_ref, gW1_ref, gb1_ref, gW2_ref, gb2_ref, gW3_ref, gb3_ref,
                 fW1_ref, fb1_ref, fW2_ref, fb2_ref,
                 sW1_ref, sb1_ref, sW2_ref, sb2_ref,
                 probs_ref, i1_ref, i2_ref, w1_ref, w2_ref,
                 fiber_ref, smooth_ref, xb_ref):
    x = x_ref[...]
    xb_ref[...] = _pack_rows(x)
    h = jnp.maximum(_bdot(x, gW1_ref[...]) + gb1_ref[...], 0.0)
    h = jnp.maximum(_bdot(h, gW2_ref[...]) + gb2_ref[...], 0.0)
    logits = _bdot(h, gW3_ref[...]) + gb3_ref[...]
    m = jnp.max(logits, axis=-1, keepdims=True)
    ex = jnp.exp(logits - m)
    p = ex / jnp.sum(ex, axis=-1, keepdims=True)
    probs_ref[...] = p

    col = jax.lax.broadcasted_iota(jnp.int32, p.shape, 1)
    v1 = jnp.max(p, axis=-1, keepdims=True)
    i1 = jnp.min(jnp.where(p == v1, col, E), axis=-1, keepdims=True)
    pm = jnp.where(col == i1, -1.0, p)
    v2 = jnp.max(pm, axis=-1, keepdims=True)
    i2 = jnp.min(jnp.where(pm == v2, col, E), axis=-1, keepdims=True)
    s = v1 + v2
    i1_ref[...] = i1
    i2_ref[...] = i2
    w1_ref[...] = v1 / s
    w2_ref[...] = v2 / s

    fh = jnp.maximum(_bdot(x, fW1_ref[...]) + fb1_ref[...], 0.0)
    fiber_ref[...] = jnp.tanh(_bdot(fh, fW2_ref[...]) + fb2_ref[...])
    sh = jnp.maximum(_bdot(x, sW1_ref[...]) + sb1_ref[...], 0.0)
    smooth_ref[...] = jax.nn.sigmoid(_bdot(sh, sW2_ref[...]) + sb2_ref[...])


def _run_gate(x, gW1, gb1, gW2, gb2, gW3, gb3, fW1, fb1, fW2, fb2,
              sW1, sb1, sW2, sb2):
    TB = 1024
    r2 = lambda b: b.reshape(1, -1)
    full = lambda shape: pl.BlockSpec(shape, lambda i: tuple(0 for _ in shape))
    return pl.pallas_call(
        _gate_kernel,
        grid=(N // TB,),
        in_specs=[
            pl.BlockSpec((TB, D), lambda i: (i, 0)),
            full((D, H)), full((1, H)), full((H, H)), full((1, H)),
            full((H, E)), full((1, E)),
            full((D, H)), full((1, H)), full((H, 1)), full((1, 1)),
            full((D, H)), full((1, H)), full((H, 1)), full((1, 1)),
        ],
        out_specs=[
            pl.BlockSpec((TB, E), lambda i: (i, 0)),
            pl.BlockSpec((TB, 1), lambda i: (i, 0)),
            pl.BlockSpec((TB, 1), lambda i: (i, 0)),
            pl.BlockSpec((TB, 1), lambda i: (i, 0)),
            pl.BlockSpec((TB, 1), lambda i: (i, 0)),
            pl.BlockSpec((TB, 1), lambda i: (i, 0)),
            pl.BlockSpec((TB, 1), lambda i: (i, 0)),
            pl.BlockSpec((TB, D // 2), lambda i: (i, 0)),
        ],
        out_shape=[
            jax.ShapeDtypeStruct((N, E), jnp.float32),
            jax.ShapeDtypeStruct((N, 1), jnp.int32),
            jax.ShapeDtypeStruct((N, 1), jnp.int32),
            jax.ShapeDtypeStruct((N, 1), jnp.float32),
            jax.ShapeDtypeStruct((N, 1), jnp.float32),
            jax.ShapeDtypeStruct((N, 1), jnp.float32),
            jax.ShapeDtypeStruct((N, 1), jnp.float32),
            jax.ShapeDtypeStruct((N, D // 2), jnp.int32),
        ],
        compiler_params=pltpu.CompilerParams(
            dimension_semantics=("parallel",)),
    )(x, gW1, r2(gb1), gW2, r2(gb2), gW3, r2(gb3),
      fW1, r2(fb1), fW2, r2(fb2), sW1, r2(sb1), sW2, r2(sb2))


def _route(i1, i2, w1, w2):
    ids = jnp.concatenate([i1[:, 0], i2[:, 0]])
    onehot = (ids[:, None] == jnp.arange(E, dtype=jnp.int32)[None, :])
    oh = onehot.astype(jnp.int32)
    ranks_inc = jnp.cumsum(oh, axis=0)
    counts = ranks_inc[-1]
    nb = (counts + B - 1) // B
    ends = jnp.cumsum(nb)
    po = jnp.concatenate([jnp.zeros((1,), jnp.int32),
                          (ends[:-1] * B).astype(jnp.int32)])
    pp = jnp.sum(jnp.where(onehot, ranks_inc - oh + po[None, :], 0),
                 axis=1).astype(jnp.int32)
    bidx = jnp.arange(NBLK, dtype=jnp.int32)
    be = jnp.minimum(
        jnp.sum((bidx[:, None] >= ends[None, :]).astype(jnp.int32), axis=1),
        E - 1).astype(jnp.int32)
    return pp, be


def _sc_dispatch(xb, pp):
    mesh = plsc.VectorSubcoreMesh(core_axis_name="c", subcore_axis_name="s")

    @functools.partial(
        pl.kernel, mesh=mesh,
        out_type=jax.ShapeDtypeStruct((Q, D // 2), jnp.int32),
        scratch_types=[pltpu.VMEM((WB,), jnp.int32),
                       pltpu.VMEM((WB, D // 2), jnp.int32)],
    )
    def dispatch(x_hbm, pp_hbm, xg_hbm, idx_v, rows_v):
        wid = lax.axis_index("s") * NC + lax.axis_index("c")
        base = wid * (N // NW)

        @pl.loop(0, N // NW, step=WB)
        def _(off):
            t = base + off
            pltpu.sync_copy(x_hbm.at[pl.ds(t, WB)], rows_v)
            pltpu.sync_copy(pp_hbm.at[pl.ds(t, WB)], idx_v)
            pltpu.sync_copy(rows_v, xg_hbm.at[idx_v])
            pltpu.sync_copy(pp_hbm.at[pl.ds(N + t, WB)], idx_v)
            pltpu.sync_copy(rows_v, xg_hbm.at[idx_v])

    return dispatch(xb, pp)


def _expert_kernel(be_ref, xg_ref, eW1a_ref, eW1b_ref, eb1_ref, eW2_ref,
                   eb2_ref, eW3_ref, eb3_ref, ys_ref):
    b0, b1 = _unpack_rows(xg_ref[...])
    h = jnp.maximum(_bdot(b0, eW1a_ref[0]) + _bdot(b1, eW1b_ref[0])
                    + eb1_ref[0], 0.0)
    h = jnp.maximum(_bdot(h, eW2_ref[0]) + eb2_ref[0], 0.0)
    o = _bdot(h, eW3_ref[0]) + eb3_ref[0]
    ys_ref[...] = _pack_rows(o)


def _run_experts(xg, be, eW1, eb1, eW2, eb2, eW3, eb3):
    grid_spec = pltpu.PrefetchScalarGridSpec(
        num_scalar_prefetch=1,
        grid=(NBLK,),
        in_specs=[
            pl.BlockSpec((B, D // 2), lambda b, be: (b, 0)),
            pl.BlockSpec((1, D // 2, H), lambda b, be: (be[b], 0, 0)),
            pl.BlockSpec((1, D // 2, H), lambda b, be: (be[b], 1, 0)),
            pl.BlockSpec((1, 1, H), lambda b, be: (be[b], 0, 0)),
            pl.BlockSpec((1, H, H), lambda b, be: (be[b], 0, 0)),
            pl.BlockSpec((1, 1, H), lambda b, be: (be[b], 0, 0)),
            pl.BlockSpec((1, H, D), lambda b, be: (be[b], 0, 0)),
            pl.BlockSpec((1, 1, D), lambda b, be: (be[b], 0, 0)),
        ],
        out_specs=pl.BlockSpec((B, D // 2), lambda b, be: (b, 0)),
    )
    return pl.pallas_call(
        _expert_kernel,
        grid_spec=grid_spec,
        out_shape=jax.ShapeDtypeStruct((Q, D // 2), jnp.int32),
        compiler_params=pltpu.CompilerParams(
            dimension_semantics=("arbitrary",)),
    )(be, xg, eW1, eW1, eb1[:, None, :], eW2, eb2[:, None, :],
      eW3, eb3[:, None, :])


def _sc_combine(ys, pp):
    mesh = plsc.VectorSubcoreMesh(core_axis_name="c", subcore_axis_name="s")

    @functools.partial(
        pl.kernel, mesh=mesh,
        out_type=jax.ShapeDtypeStruct((A, D // 2), jnp.int32),
        scratch_types=[pltpu.VMEM((A // NW,), jnp.int32),
                       pltpu.VMEM((CWB, D // 2), jnp.int32),
                       pltpu.VMEM((CWB, D // 2), jnp.int32),
                       pltpu.SemaphoreType.DMA,
                       pltpu.SemaphoreType.DMA],
    )
    def combine(ys_hbm, pp_hbm, g_hbm, idx_v, rows0_v, rows1_v, sem0, sem1):
        wid = lax.axis_index("s") * NC + lax.axis_index("c")
        base = wid * (A // NW)
        nw_ = (A // NW) // CWB
        pltpu.sync_copy(pp_hbm.at[pl.ds(base, A // NW)], idx_v)
        bufs = [(rows0_v, sem0), (rows1_v, sem1)]
        cps = [None] * nw_
        for w in range(nw_):
            rv, sm = bufs[w % 2]
            if w >= 2:
                cps[w - 2].wait()
                pltpu.sync_copy(rv, g_hbm.at[pl.ds(base + (w - 2) * CWB, CWB)])
            cps[w] = pltpu.async_copy(
                ys_hbm.at[idx_v.at[pl.ds(w * CWB, CWB)]], rv, sem=sm)
        for w in range(nw_ - 2, nw_):
            rv, _ = bufs[w % 2]
            cps[w].wait()
            pltpu.sync_copy(rv, g_hbm.at[pl.ds(base + w * CWB, CWB)])

    return combine(ys, pp)


def _add_kernel(g0_ref, g1_ref, w1_ref, w2_ref, out_ref):
    a0, a1 = _unpack_rows(g0_ref[...])
    b0, b1 = _unpack_rows(g1_ref[...])
    w1 = w1_ref[...]
    w2 = w2_ref[...]
    out_ref[:, :D // 2] = a0 * w1 + b0 * w2
    out_ref[:, D // 2:] = a1 * w1 + b1 * w2


def _run_add(g, w1, w2):
    TB = 1024
    nb = N // TB
    return pl.pallas_call(
        _add_kernel,
        grid=(nb,),
        in_specs=[
            pl.BlockSpec((TB, D // 2), lambda i: (i, 0)),
            pl.BlockSpec((TB, D // 2), lambda i: (i + nb, 0)),
            pl.BlockSpec((TB, 1), lambda i: (i, 0)),
            pl.BlockSpec((TB, 1), lambda i: (i, 0)),
        ],
        out_specs=pl.BlockSpec((TB, D), lambda i: (i, 0)),
        out_shape=jax.ShapeDtypeStruct((N, D), jnp.float32),
        compiler_params=pltpu.CompilerParams(
            dimension_semantics=("parallel",)),
    )(g, g, w1, w2)


def kernel(x, eW1, eb1, eW2, eb2, eW3, eb3, gW1, gb1, gW2, gb2, gW3, gb3,
           fW1, fb1, fW2, fb2, sW1, sb1, sW2, sb2):
    (gate_probs, i1, i2, w1, w2, fiber, smooth, xb) = _run_gate(
        x, gW1, gb1, gW2, gb2, gW3, gb3, fW1, fb1, fW2, fb2,
        sW1, sb1, sW2, sb2)

    pp, be = _route(i1, i2, w1, w2)
    xg = _sc_dispatch(xb, pp)
    ys = _run_experts(xg, be, eW1, eb1, eW2, eb2, eW3, eb3)
    g = _sc_combine(ys, pp)
    out = _run_add(g, w1, w2)
    return (out, gate_probs, fiber, smooth)


# speedup vs baseline: 1.0486x; 1.0028x over previous
import functools

import jax
import jax.numpy as jnp
from jax import lax
from jax.experimental import pallas as pl
from jax.experimental.pallas import tpu as pltpu
from jax.experimental.pallas import tpu_sc as plsc

N, D, H, E = 8192, 1024, 512, 8
A = 2 * N
B = 512
Q = A + E * B
NBLK = Q // B

NC, NS = 2, 16
NW = NC * NS
WB = 128
CWB = 64


def _bdot(a, b):
    return jnp.dot(a.astype(jnp.bfloat16), b.astype(jnp.bfloat16),
                   preferred_element_type=jnp.float32)


_HI_MASK = 0xFFFF0000


def _pack_rows(v):
    r = v.astype(jnp.bfloat16).astype(jnp.float32)
    hw = r.shape[-1] // 2
    u0 = lax.bitcast_convert_type(r[:, :hw], jnp.uint32) >> 16
    u1 = lax.bitcast_convert_type(r[:, hw:], jnp.uint32) & jnp.uint32(_HI_MASK)
    return lax.bitcast_convert_type(u0 | u1, jnp.int32)


def _unpack_rows(p):
    pu = lax.bitcast_convert_type(p, jnp.uint32)
    b0 = lax.bitcast_convert_type(pu << 16, jnp.float32)
    b1 = lax.bitcast_convert_type(pu & jnp.uint32(_HI_MASK), jnp.float32)
    return b0, b1


def _gate_kernel(x_ref, gW1_ref, gb1_ref, gW2_ref, gb2_ref, gW3_ref, gb3_ref,
                 fW1_ref, fb1_ref, fW2_ref, fb2_ref,
                 sW1_ref, sb1_ref, sW2_ref, sb2_ref,
                 probs_ref, i1_ref, i2_ref, w1_ref, w2_ref,
                 fiber_ref, smooth_ref, xb_ref):
    x = x_ref[...]
    xb_ref[...] = _pack_rows(x)
    h = jnp.maximum(_bdot(x, gW1_ref[...]) + gb1_ref[...], 0.0)
    h = jnp.maximum(_bdot(h, gW2_ref[...]) + gb2_ref[...], 0.0)
    logits = _bdot(h, gW3_ref[...]) + gb3_ref[...]
    m = jnp.max(logits, axis=-1, keepdims=True)
    ex = jnp.exp(logits - m)
    p = ex / jnp.sum(ex, axis=-1, keepdims=True)
    probs_ref[...] = p

    col = jax.lax.broadcasted_iota(jnp.int32, p.shape, 1)
    v1 = jnp.max(p, axis=-1, keepdims=True)
    i1 = jnp.min(jnp.where(p == v1, col, E), axis=-1, keepdims=True)
    pm = jnp.where(col == i1, -1.0, p)
    v2 = jnp.max(pm, axis=-1, keepdims=True)
    i2 = jnp.min(jnp.where(pm == v2, col, E), axis=-1, keepdims=True)
    s = v1 + v2
    i1_ref[...] = i1
    i2_ref[...] = i2
    w1_ref[...] = v1 / s
    w2_ref[...] = v2 / s

    fh = jnp.maximum(_bdot(x, fW1_ref[...]) + fb1_ref[...], 0.0)
    fiber_ref[...] = jnp.tanh(_bdot(fh, fW2_ref[...]) + fb2_ref[...])
    sh = jnp.maximum(_bdot(x, sW1_ref[...]) + sb1_ref[...], 0.0)
    smooth_ref[...] = jax.nn.sigmoid(_bdot(sh, sW2_ref[...]) + sb2_ref[...])


def _run_gate(x, gW1, gb1, gW2, gb2, gW3, gb3, fW1, fb1, fW2, fb2,
              sW1, sb1, sW2, sb2):
    TB = 1024
    r2 = lambda b: b.reshape(1, -1)
    full = lambda shape: pl.BlockSpec(shape, lambda i: tuple(0 for _ in shape))
    return pl.pallas_call(
        _gate_kernel,
        grid=(N // TB,),
        in_specs=[
            pl.BlockSpec((TB, D), lambda i: (i, 0)),
            full((D, H)), full((1, H)), full((H, H)), full((1, H)),
            full((H, E)), full((1, E)),
            full((D, H)), full((1, H)), full((H, 1)), full((1, 1)),
            full((D, H)), full((1, H)), full((H, 1)), full((1, 1)),
        ],
        out_specs=[
            pl.BlockSpec((TB, E), lambda i: (i, 0)),
            pl.BlockSpec((TB, 1), lambda i: (i, 0)),
            pl.BlockSpec((TB, 1), lambda i: (i, 0)),
            pl.BlockSpec((TB, 1), lambda i: (i, 0)),
            pl.BlockSpec((TB, 1), lambda i: (i, 0)),
            pl.BlockSpec((TB, 1), lambda i: (i, 0)),
            pl.BlockSpec((TB, 1), lambda i: (i, 0)),
            pl.BlockSpec((TB, D // 2), lambda i: (i, 0)),
        ],
        out_shape=[
            jax.ShapeDtypeStruct((N, E), jnp.float32),
            jax.ShapeDtypeStruct((N, 1), jnp.int32),
            jax.ShapeDtypeStruct((N, 1), jnp.int32),
            jax.ShapeDtypeStruct((N, 1), jnp.float32),
            jax.ShapeDtypeStruct((N, 1), jnp.float32),
            jax.ShapeDtypeStruct((N, 1), jnp.float32),
            jax.ShapeDtypeStruct((N, 1), jnp.float32),
            jax.ShapeDtypeStruct((N, D // 2), jnp.int32),
        ],
        compiler_params=pltpu.CompilerParams(
            dimension_semantics=("parallel",)),
    )(x, gW1, r2(gb1), gW2, r2(gb2), gW3, r2(gb3),
      fW1, r2(fb1), fW2, r2(fb2), sW1, r2(sb1), sW2, r2(sb2))


def _route(i1, i2, w1, w2):
    ids = jnp.concatenate([i1[:, 0], i2[:, 0]])
    onehot = (ids[:, None] == jnp.arange(E, dtype=jnp.int32)[None, :])
    oh = onehot.astype(jnp.int16)
    ranks_inc = jnp.cumsum(oh, axis=0)
    counts = ranks_inc[-1].astype(jnp.int32)
    nb = (counts + B - 1) // B
    ends = jnp.cumsum(nb)
    po = jnp.concatenate([jnp.zeros((1,), jnp.int32),
                          (ends[:-1] * B).astype(jnp.int32)])
    pp = jnp.sum(jnp.where(onehot, ranks_inc - oh + po[None, :], 0),
                 axis=1).astype(jnp.int32)
    bidx = jnp.arange(NBLK, dtype=jnp.int32)
    be = jnp.minimum(
        jnp.sum((bidx[:, None] >= ends[None, :]).astype(jnp.int32), axis=1),
        E - 1).astype(jnp.int32)
    return pp, be


def _sc_dispatch(xb, pp):
    mesh = plsc.VectorSubcoreMesh(core_axis_name="c", subcore_axis_name="s")

    @functools.partial(
        pl.kernel, mesh=mesh,
        out_type=jax.ShapeDtypeStruct((Q, D // 2), jnp.int32),
        scratch_types=[pltpu.VMEM((WB,), jnp.int32),
                       pltpu.VMEM((WB, D // 2), jnp.int32)],
    )
    def dispatch(x_hbm, pp_hbm, xg_hbm, idx_v, rows_v):
        wid = lax.axis_index("s") * NC + lax.axis_index("c")
        base = wid * (N // NW)

        @pl.loop(0, N // NW, step=WB)
        def _(off):
            t = base + off
            pltpu.sync_copy(x_hbm.at[pl.ds(t, WB)], rows_v)
            pltpu.sync_copy(pp_hbm.at[pl.ds(t, WB)], idx_v)
            pltpu.sync_copy(rows_v, xg_hbm.at[idx_v])
            pltpu.sync_copy(pp_hbm.at[pl.ds(N + t, WB)], idx_v)
            pltpu.sync_copy(rows_v, xg_hbm.at[idx_v])

    return dispatch(xb, pp)


def _expert_kernel(be_ref, xg_ref, eW1a_ref, eW1b_ref, eb1_ref, eW2_ref,
                   eb2_ref, eW3_ref, eb3_ref, ys_ref):
    b0, b1 = _unpack_rows(xg_ref[...])
    h = jnp.maximum(_bdot(b0, eW1a_ref[0]) + _bdot(b1, eW1b_ref[0])
                    + eb1_ref[0], 0.0)
    h = jnp.maximum(_bdot(h, eW2_ref[0]) + eb2_ref[0], 0.0)
    o = _bdot(h, eW3_ref[0]) + eb3_ref[0]
    ys_ref[...] = _pack_rows(o)


def _run_experts(xg, be, eW1, eb1, eW2, eb2, eW3, eb3):
    grid_spec = pltpu.PrefetchScalarGridSpec(
        num_scalar_prefetch=1,
        grid=(NBLK,),
        in_specs=[
            pl.BlockSpec((B, D // 2), lambda b, be: (b, 0)),
            pl.BlockSpec((1, D // 2, H), lambda b, be: (be[b], 0, 0)),
            pl.BlockSpec((1, D // 2, H), lambda b, be: (be[b], 1, 0)),
            pl.BlockSpec((1, 1, H), lambda b, be: (be[b], 0, 0)),
            pl.BlockSpec((1, H, H), lambda b, be: (be[b], 0, 0)),
            pl.BlockSpec((1, 1, H), lambda b, be: (be[b], 0, 0)),
            pl.BlockSpec((1, H, D), lambda b, be: (be[b], 0, 0)),
            pl.BlockSpec((1, 1, D), lambda b, be: (be[b], 0, 0)),
        ],
        out_specs=pl.BlockSpec((B, D // 2), lambda b, be: (b, 0)),
    )
    return pl.pallas_call(
        _expert_kernel,
        grid_spec=grid_spec,
        out_shape=jax.ShapeDtypeStruct((Q, D // 2), jnp.int32),
        compiler_params=pltpu.CompilerParams(
            dimension_semantics=("arbitrary",)),
    )(be, xg, eW1, eW1, eb1[:, None, :], eW2, eb2[:, None, :],
      eW3, eb3[:, None, :])


def _sc_combine(ys, pp):
    mesh = plsc.VectorSubcoreMesh(core_axis_name="c", subcore_axis_name="s")

    @functools.partial(
        pl.kernel, mesh=mesh,
        out_type=jax.ShapeDtypeStruct((A, D // 2), jnp.int32),
        scratch_types=[pltpu.VMEM((A // NW,), jnp.int32),
                       pltpu.VMEM((CWB, D // 2), jnp.int32),
                       pltpu.VMEM((CWB, D // 2), jnp.int32),
                       pltpu.SemaphoreType.DMA,
                       pltpu.SemaphoreType.DMA],
    )
    def combine(ys_hbm, pp_hbm, g_hbm, idx_v, rows0_v, rows1_v, sem0, sem1):
        wid = lax.axis_index("s") * NC + lax.axis_index("c")
        base = wid * (A // NW)
        nw_ = (A // NW) // CWB
        pltpu.sync_copy(pp_hbm.at[pl.ds(base, A // NW)], idx_v)
        bufs = [(rows0_v, sem0), (rows1_v, sem1)]
        cps = [None] * nw_
        for w in range(nw_):
            rv, sm = bufs[w % 2]
            if w >= 2:
                cps[w - 2].wait()
                pltpu.sync_copy(rv, g_hbm.at[pl.ds(base + (w - 2) * CWB, CWB)])
            cps[w] = pltpu.async_copy(
                ys_hbm.at[idx_v.at[pl.ds(w * CWB, CWB)]], rv, sem=sm)
        for w in range(nw_ - 2, nw_):
            rv, _ = bufs[w % 2]
            cps[w].wait()
            pltpu.sync_copy(rv, g_hbm.at[pl.ds(base + w * CWB, CWB)])

    return combine(ys, pp)


def _add_kernel(g0_ref, g1_ref, w1_ref, w2_ref, out_ref):
    a0, a1 = _unpack_rows(g0_ref[...])
    b0, b1 = _unpack_rows(g1_ref[...])
    w1 = w1_ref[...]
    w2 = w2_ref[...]
    out_ref[:, :D // 2] = a0 * w1 + b0 * w2
    out_ref[:, D // 2:] = a1 * w1 + b1 * w2


def _run_add(g, w1, w2):
    TB = 1024
    nb = N // TB
    return pl.pallas_call(
        _add_kernel,
        grid=(nb,),
        in_specs=[
            pl.BlockSpec((TB, D // 2), lambda i: (i, 0)),
            pl.BlockSpec((TB, D // 2), lambda i: (i + nb, 0)),
            pl.BlockSpec((TB, 1), lambda i: (i, 0)),
            pl.BlockSpec((TB, 1), lambda i: (i, 0)),
        ],
        out_specs=pl.BlockSpec((TB, D), lambda i: (i, 0)),
        out_shape=jax.ShapeDtypeStruct((N, D), jnp.float32),
        compiler_params=pltpu.CompilerParams(
            dimension_semantics=("parallel",)),
    )(g, g, w1, w2)


def kernel(x, eW1, eb1, eW2, eb2, eW3, eb3, gW1, gb1, gW2, gb2, gW3, gb3,
           fW1, fb1, fW2, fb2, sW1, sb1, sW2, sb2):
    (gate_probs, i1, i2, w1, w2, fiber, smooth, xb) = _run_gate(
        x, gW1, gb1, gW2, gb2, gW3, gb3, fW1, fb1, fW2, fb2,
        sW1, sb1, sW2, sb2)

    pp, be = _route(i1, i2, w1, w2)
    xg = _sc_dispatch(xb, pp)
    ys = _run_experts(xg, be, eW1, eb1, eW2, eb2, eW3, eb3)
    g = _sc_combine(ys, pp)
    out = _run_add(g, w1, w2)
    return (out, gate_probs, fiber, smooth)
